# baseline (speedup 1.0000x reference)
import functools

import jax
import jax.numpy as jnp
from jax import lax
from jax.experimental import pallas as pl
from jax.experimental.pallas import tpu as pltpu
from jax.experimental.pallas import tpu_sc as plsc
from jax.experimental.layout import Format, Layout

NC = 2
NS = 16
L = 16
NW = NC * NS

B = 16384
D = 64
BPW = B // NW
HALF = 256
GROUPS_PER_STEP = 4
ROWS_PER_STEP = GROUPS_PER_STEP * L
N_STEPS = HALF // ROWS_PER_STEP
PV = 3072


def _sc_body(idx2u_hbm, idx2m_hbm, rat_hbm, ctab_hbm,
             params_hbm, out_hbm,
             idx2u_v, idx2m_v, rowsA, rowsB, rat_v,
             out1d, out_pl, params_v, semA, semB):
    wid = lax.axis_index("s") * NC + lax.axis_index("c")

    pltpu.sync_copy(params_hbm, params_v)
    pltpu.sync_copy(idx2u_hbm.at[wid], idx2u_v)
    pltpu.sync_copy(idx2m_hbm.at[wid], idx2m_v)
    for j in range(4):
        pltpu.sync_copy(rat_hbm.at[wid].at[j], rat_v.at[pl.ds(j * 128, 128)])

    def fire(tab, idx_v, j0, rows, sem):
        c0 = pltpu.async_copy(tab.at[idx_v.at[j0]],
                              rows.at[pl.ds(0, 128)], sem)
        c1 = pltpu.async_copy(tab.at[idx_v.at[j0 + 1]],
                              rows.at[pl.ds(128, 128)], sem)
        return c0, c1

    iota = lax.iota(jnp.int32, L)
    zero = jnp.zeros((L,), jnp.float32)
    bias = params_v[pl.ds(2 * D * L, L)]

    def wvec(d):
        return params_v[pl.ds(d * L, L)]

    def batch(rows, phase, poff, coff, first, lacc_in):
        def step(c, lacc):
            base = c * ROWS_PER_STEP
            ids = [base + q * L + iota for q in range(GROUPS_PER_STEP)]
            if first:
                accs = [bias for _ in range(GROUPS_PER_STEP)]
            else:
                accs = [out1d[pl.ds(phase * HALF + base + q * L, L)]
                        for q in range(GROUPS_PER_STEP)]
            for d in range(D):
                w = wvec(poff + d)
                dcol = jnp.full((L,), coff + d, jnp.int32)
                for q in range(GROUPS_PER_STEP):
                    accs[q] = accs[q] + plsc.load_gather(
                        rows, [ids[q], dcol]) * w
            for q in range(GROUPS_PER_STEP):
                off = phase * HALF + base + q * L
                out1d[pl.ds(off, L)] = accs[q]
                if not first:
                    diff = accs[q] - rat_v[pl.ds(off, L)]
                    lacc = lacc + diff * diff
            return lacc

        return lax.fori_loop(0, N_STEPS, step, lacc_in, unroll=False)

    u0 = fire(ctab_hbm, idx2u_v, 0, rowsA, semA)
    u1 = fire(ctab_hbm, idx2u_v, 2, rowsB, semB)
    u0[0].wait(); u0[1].wait()
    batch(rowsA, 0, 0, 0, True, zero)
    m0 = fire(ctab_hbm, idx2m_v, 0, rowsA, semA)
    u1[0].wait(); u1[1].wait()
    batch(rowsB, 1, 0, 0, True, zero)
    m1 = fire(ctab_hbm, idx2m_v, 2, rowsB, semB)
    m0[0].wait(); m0[1].wait()
    lacc = batch(rowsA, 0, D, D, False, zero)
    m1[0].wait(); m1[1].wait()
    lacc = batch(rowsB, 1, D, D, False, lacc)

    for j in range(4):
        pltpu.sync_copy(out1d.at[pl.ds(j * 128, 128)], out_hbm.at[wid].at[j])
    z16 = jnp.zeros((L,), jnp.float32)
    for k in range(8):
        out_pl[pl.ds(k * L, L)] = lacc if k == 0 else z16
    pltpu.sync_copy(out_pl, out_hbm.at[wid].at[4])


@jax.jit
def _run(idx2u, idx2m, rat3, ctab, params_bc):
    mesh = plsc.VectorSubcoreMesh(core_axis_name="c", subcore_axis_name="s",
                                  num_cores=NC, num_subcores=NS)
    out3, = pl.kernel(
        _sc_body,
        out_type=[jax.ShapeDtypeStruct((NW, 8, 128), jnp.float32)],
        mesh=mesh,
        compiler_params=pltpu.CompilerParams(
            needs_layout_passes=False, use_tc_tiling_on_sc=True),
        scratch_types=[
            pltpu.VMEM((8, 128), jnp.int32),
            pltpu.VMEM((8, 128), jnp.int32),
            pltpu.VMEM((HALF, 128), jnp.float32),
            pltpu.VMEM((HALF, 128), jnp.float32),
            pltpu.VMEM((BPW,), jnp.float32),
            pltpu.VMEM((BPW,), jnp.float32),
            pltpu.VMEM((128,), jnp.float32),
            pltpu.VMEM((PV,), jnp.float32),
            pltpu.SemaphoreType.DMA,
            pltpu.SemaphoreType.DMA,
        ],
    )(idx2u, idx2m, rat3, ctab, params_bc)
    output = out3[:, :4, :].reshape(B, 1)
    loss = jnp.sum(out3[:, 4, :]) * (1.0 / B)
    return output, loss


def kernel(users, movies, ratings, user_table, movie_table, W, b):
    idx2u = jnp.pad(users.reshape(NW, 4, 128), ((0, 0), (0, 4), (0, 0)))
    idx2m = jnp.pad(movies.reshape(NW, 4, 128), ((0, 0), (0, 4), (0, 0)))
    rat3 = jnp.pad(ratings.reshape(NW, 4, 128), ((0, 0), (0, 4), (0, 0)))
    ctab = jnp.concatenate(
        [user_table,
         jnp.pad(movie_table, ((0, user_table.shape[0] - movie_table.shape[0]),
                               (0, 0)))], axis=1)
    params = jnp.concatenate(
        [W.reshape(2 * D), b.reshape(1), jnp.zeros((7,), jnp.float32)])
    params_bc = jnp.pad(
        jnp.broadcast_to(params[:, None], (2 * D + 8, L)).reshape(-1),
        (0, PV - (2 * D + 8) * L))
    return _run(idx2u, idx2m, rat3, ctab, params_bc)
